# Initial kernel scaffold; baseline (speedup 1.0000x reference)
import jax
import jax.numpy as jnp
from jax.experimental import pallas as pl


def kernel(input_labes, out_labels, num_sampled, in_embed, out_embed):
    raise NotImplementedError("write your pallas kernel here")


# speedup vs baseline: 2.7885x; 2.7885x over previous
import functools

import jax
import jax.numpy as jnp
from jax import lax
from jax.experimental import pallas as pl
from jax.experimental.pallas import tpu as pltpu
from jax.experimental.pallas import tpu_sc as plsc

_NUM_CLASSES = 100000
_D = 128
_B = 1024
_W = 20
_S = 16
_N = _B * _W
_NC = 2
_NSC = 16
_NW = _NC * _NSC
_RPT = _N // _NW
_RB = 16
_NBLK = _RPT // _RB
_L = 16

_TC_STEPS = 20


def _sc_scores(in_tab, out_tab, iidx_h, oidx_h, nidx_h, posp_h, negp_h,
               iidx_v, oidx_v, nidx_v, inp_v, out_v, noise_a, noise_b,
               posp_sc, negp_sc, sem0, sem1, sem2, sem3):
    cid = lax.axis_index("c")
    sid = lax.axis_index("s")
    wid = sid * _NC + cid
    base0 = wid * _RPT

    def block_body(blk, carry):
        base = base0 + blk * _RB
        pltpu.sync_copy(iidx_h.at[pl.ds(base, _RB)], iidx_v)
        pltpu.sync_copy(oidx_h.at[pl.ds(base, _RB)], oidx_v)
        pltpu.sync_copy(nidx_h.at[pl.ds(base * _S, _RB * _S)], nidx_v)
        cp0 = pltpu.async_copy(in_tab.at[iidx_v], inp_v, sem0)
        cp1 = pltpu.async_copy(out_tab.at[oidx_v], out_v, sem1)
        cp2 = pltpu.async_copy(out_tab.at[nidx_v.at[pl.ds(0, 128)]], noise_a, sem2)
        cp3 = pltpu.async_copy(out_tab.at[nidx_v.at[pl.ds(128, 128)]], noise_b, sem3)
        cp0.wait()
        cp1.wait()
        cp2.wait()
        cp3.wait()

        for r in range(_RB):
            ich = [inp_v[r, pl.ds(k * _L, _L)] for k in range(_D // _L)]
            nbuf = noise_a if r < 8 else noise_b
            for s in range(_S):
                j = (r % 8) * _S + s
                acc = nbuf[j, pl.ds(0, _L)] * ich[0]
                for k in range(1, _D // _L):
                    acc = acc + nbuf[j, pl.ds(k * _L, _L)] * ich[k]
                negp_sc[pl.ds((r * _S + s) * _L, _L)] = -acc
            pacc = out_v[r, pl.ds(0, _L)] * ich[0]
            for k in range(1, _D // _L):
                pacc = pacc + out_v[r, pl.ds(k * _L, _L)] * ich[k]
            posp_sc[pl.ds(r * _L, _L)] = pacc

        pltpu.sync_copy(posp_sc, posp_h.at[pl.ds(base * _L, _RB * _L)])
        pltpu.sync_copy(negp_sc, negp_h.at[pl.ds(base * _S * _L, _RB * _S * _L)])
        return carry

    lax.fori_loop(0, _NBLK, block_body, 0)


def _tc_loss(posp_ref, negp_ref, maskp_ref, maskn_ref, g_ref, out_ref):
    t = pl.program_id(0)

    def logsig(x):
        return jnp.minimum(x, 0.0) - jnp.log1p(jnp.exp(-jnp.abs(x)))

    g = g_ref[...]
    yn = jax.lax.dot(negp_ref[...], g, precision=jax.lax.Precision.HIGHEST)
    yp = jax.lax.dot(posp_ref[...], g, precision=jax.lax.Precision.HIGHEST)
    mn = jnp.tile(maskn_ref[...], (yn.shape[0] // 2, 1))
    contrib = jnp.sum(logsig(yn) * mn) + jnp.sum(logsig(yp) * maskp_ref[...])

    @pl.when(t == 0)
    def _init():
        out_ref[...] = jnp.zeros((1, 1), jnp.float32)

    out_ref[...] = out_ref[...] + jnp.full((1, 1), contrib, jnp.float32)

    @pl.when(t == _TC_STEPS - 1)
    def _fin():
        out_ref[...] = out_ref[...] * (-1.0 / _B)


def kernel(input_labes, out_labels, num_sampled, in_embed, out_embed):
    inp_idx = jnp.tile(input_labes, _W).astype(jnp.int32)
    out_idx = out_labels.reshape(-1).astype(jnp.int32)
    noise_idx = jax.random.randint(jax.random.key(42), (_N, _S),
                                   0, _NUM_CLASSES - 1).astype(jnp.int32)
    noise_flat = noise_idx.reshape(-1)

    mesh = plsc.VectorSubcoreMesh(core_axis_name="c", subcore_axis_name="s")
    sc = functools.partial(
        pl.kernel, mesh=mesh,
        out_type=[jax.ShapeDtypeStruct((_N * _L,), jnp.float32),
                  jax.ShapeDtypeStruct((_N * _S * _L,), jnp.float32)],
        scratch_types=[
            pltpu.VMEM((_RB,), jnp.int32),
            pltpu.VMEM((_RB,), jnp.int32),
            pltpu.VMEM((_RB * _S,), jnp.int32),
            pltpu.VMEM((_RB, _D), jnp.float32),
            pltpu.VMEM((_RB, _D), jnp.float32),
            pltpu.VMEM((128, _D), jnp.float32),
            pltpu.VMEM((128, _D), jnp.float32),
            pltpu.VMEM((_RB * _L,), jnp.float32),
            pltpu.VMEM((_RB * _S * _L,), jnp.float32),
            pltpu.SemaphoreType.DMA,
            pltpu.SemaphoreType.DMA,
            pltpu.SemaphoreType.DMA,
            pltpu.SemaphoreType.DMA,
        ],
    )(_sc_scores)
    posp, negp = sc(in_embed, out_embed, inp_idx, out_idx, noise_flat)

    posp2 = posp.reshape(_N * _L // 128, 128)
    negp2 = negp.reshape(_N * _S * _L // 128, 128)

    gi = jnp.arange(128) // _L
    g = (gi[:, None] == gi[None, :]).astype(jnp.float32)

    col = jnp.arange(128)
    once = (col % _L == 0).astype(jnp.float32)
    maskp = once.reshape(1, 128)
    srow = jnp.arange(2)
    sample_id = 8 * srow[:, None] + (col[None, :] // _L)
    maskn = (once[None, :] * (sample_id < num_sampled)).astype(jnp.float32)

    loss = pl.pallas_call(
        _tc_loss,
        grid=(_TC_STEPS,),
        in_specs=[
            pl.BlockSpec((_N * _L // 128 // _TC_STEPS, 128), lambda t: (t, 0)),
            pl.BlockSpec((_N * _S * _L // 128 // _TC_STEPS, 128), lambda t: (t, 0)),
            pl.BlockSpec((1, 128), lambda t: (0, 0)),
            pl.BlockSpec((2, 128), lambda t: (0, 0)),
            pl.BlockSpec((128, 128), lambda t: (0, 0)),
        ],
        out_specs=pl.BlockSpec((1, 1), lambda t: (0, 0)),
        out_shape=jax.ShapeDtypeStruct((1, 1), jnp.float32),
    )(posp2, negp2, maskp, maskn, g)
    return loss[0, 0]


# speedup vs baseline: 2.9468x; 1.0568x over previous
import functools

import jax
import jax.numpy as jnp
from jax import lax
from jax.experimental import pallas as pl
from jax.experimental.pallas import tpu as pltpu
from jax.experimental.pallas import tpu_sc as plsc

_NUM_CLASSES = 100000
_D = 128
_B = 1024
_W = 20
_S = 16
_N = _B * _W
_NC = 2
_NSC = 16
_NW = _NC * _NSC
_RPT = _N // _NW
_RB = 16
_NBLK = _RPT // _RB
_L = 16
_BLK_W = (_S + 1) * _RB * _L
_NBLK_G = _N // _RB

_TC_STEPS = 20
_TC_ROWS = _NBLK_G * _BLK_W // 128 // _TC_STEPS


def _sc_scores(in_tab, out_tab, comb_h, nidx_h, scp_h,
               comb_v, nidx_v, iv_v, ov_v, na_v, nb_v, scp_v,
               gsem_io, gsem_ov, gsem_na, gsem_nb, ssem):
    cid = lax.axis_index("c")
    sid = lax.axis_index("s")
    wid = sid * _NC + cid

    pltpu.sync_copy(comb_h.at[pl.ds(wid * (_NBLK * 2 * _RB), _NBLK * 2 * _RB)],
                    comb_v)
    pltpu.sync_copy(nidx_h.at[pl.ds(wid * (_RPT * _S), _RPT * _S)], nidx_v)

    def issue_gathers(g, par):
        pltpu.async_copy(in_tab.at[comb_v.at[pl.ds(g * 2 * _RB, _RB)]],
                         iv_v.at[par], gsem_io)
        pltpu.async_copy(out_tab.at[comb_v.at[pl.ds(g * 2 * _RB + _RB, _RB)]],
                         ov_v.at[par], gsem_ov)
        pltpu.async_copy(out_tab.at[nidx_v.at[pl.ds(g * _RB * _S, 128)]],
                         na_v.at[par], gsem_na)
        pltpu.async_copy(out_tab.at[nidx_v.at[pl.ds(g * _RB * _S + 128, 128)]],
                         nb_v.at[par], gsem_nb)

    issue_gathers(0, 0)

    def block_body(g, carry):
        p = lax.rem(g, 2)
        q = 1 - p
        pltpu.make_async_copy(in_tab.at[pl.ds(0, _RB)], iv_v.at[p],
                              gsem_io).wait()
        pltpu.make_async_copy(out_tab.at[pl.ds(0, _RB)], ov_v.at[p],
                              gsem_ov).wait()
        pltpu.make_async_copy(out_tab.at[pl.ds(0, 128)], na_v.at[p],
                              gsem_na).wait()
        pltpu.make_async_copy(out_tab.at[pl.ds(0, 128)], nb_v.at[p],
                              gsem_nb).wait()

        @pl.when(g + 1 < _NBLK)
        def _prefetch():
            issue_gathers(g + 1, q)

        @pl.when(g >= 1)
        def _drain():
            pltpu.make_async_copy(scp_v.at[0], scp_h.at[pl.ds(0, _BLK_W)],
                                  ssem).wait()

        for r in range(_RB):
            ich = [iv_v[p, r, pl.ds(k * _L, _L)] for k in range(_D // _L)]
            nbuf = na_v if r < 8 else nb_v
            for s in range(_S):
                j = (r % 8) * _S + s
                acc = nbuf[p, j, pl.ds(0, _L)] * ich[0]
                for k in range(1, _D // _L):
                    acc = acc + nbuf[p, j, pl.ds(k * _L, _L)] * ich[k]
                scp_v[p, pl.ds((r * _S + s) * _L, _L)] = -acc
            pacc = ov_v[p, r, pl.ds(0, _L)] * ich[0]
            for k in range(1, _D // _L):
                pacc = pacc + ov_v[p, r, pl.ds(k * _L, _L)] * ich[k]
            scp_v[p, pl.ds(_RB * _S * _L + r * _L, _L)] = pacc

        gb = wid * _NBLK + g
        pltpu.async_copy(scp_v.at[p], scp_h.at[pl.ds(gb * _BLK_W, _BLK_W)],
                         ssem)
        return carry

    lax.fori_loop(0, _NBLK, block_body, 0)
    pltpu.make_async_copy(scp_v.at[1], scp_h.at[pl.ds(0, _BLK_W)], ssem).wait()


def _tc_loss(scp_ref, mask_ref, g_ref, out_ref):
    t = pl.program_id(0)

    def logsig(x):
        return jnp.minimum(x, 0.0) - jnp.log1p(jnp.exp(-jnp.abs(x)))

    y = jax.lax.dot(scp_ref[...], g_ref[...],
                    precision=jax.lax.Precision.HIGHEST)
    contrib = jnp.sum(logsig(y) * mask_ref[...])

    @pl.when(t == 0)
    def _init():
        out_ref[...] = jnp.zeros((1, 1), jnp.float32)

    out_ref[...] = out_ref[...] + jnp.full((1, 1), contrib, jnp.float32)

    @pl.when(t == _TC_STEPS - 1)
    def _fin():
        out_ref[...] = out_ref[...] * (-1.0 / _B)


def kernel(input_labes, out_labels, num_sampled, in_embed, out_embed):
    inp_idx = jnp.tile(input_labes, _W).astype(jnp.int32)
    out_idx = out_labels.reshape(-1).astype(jnp.int32)
    comb = jnp.concatenate([inp_idx.reshape(_NBLK_G, _RB),
                            out_idx.reshape(_NBLK_G, _RB)],
                           axis=1).reshape(-1)
    noise_idx = jax.random.randint(jax.random.key(42), (_N, _S),
                                   0, _NUM_CLASSES - 1).astype(jnp.int32)
    noise_flat = noise_idx.reshape(-1)

    mesh = plsc.VectorSubcoreMesh(core_axis_name="c", subcore_axis_name="s")
    sc = functools.partial(
        pl.kernel, mesh=mesh,
        out_type=[jax.ShapeDtypeStruct((_NBLK_G * _BLK_W,), jnp.float32)],
        scratch_types=[
            pltpu.VMEM((_NBLK * 2 * _RB,), jnp.int32),
            pltpu.VMEM((_RPT * _S,), jnp.int32),
            pltpu.VMEM((2, _RB, _D), jnp.float32),
            pltpu.VMEM((2, _RB, _D), jnp.float32),
            pltpu.VMEM((2, 128, _D), jnp.float32),
            pltpu.VMEM((2, 128, _D), jnp.float32),
            pltpu.VMEM((2, _BLK_W), jnp.float32),
            pltpu.SemaphoreType.DMA,
            pltpu.SemaphoreType.DMA,
            pltpu.SemaphoreType.DMA,
            pltpu.SemaphoreType.DMA,
            pltpu.SemaphoreType.DMA,
        ],
    )(_sc_scores)
    (scp,) = sc(in_embed, out_embed, comb, noise_flat)
    scp2 = scp.reshape(_NBLK_G * _BLK_W // 128, 128)

    gi = jnp.arange(128) // _L
    g = (gi[:, None] == gi[None, :]).astype(jnp.float32)

    row = jnp.arange(34)
    col = jnp.arange(128)
    once = (col % _L == 0)[None, :]
    sid = 8 * (row[:, None] % 2) + col[None, :] // _L
    m34 = jnp.where(row[:, None] < 32, once & (sid < num_sampled), once)
    mask_full = jnp.tile(m34.astype(jnp.float32), (_TC_ROWS // 34, 1))

    loss = pl.pallas_call(
        _tc_loss,
        grid=(_TC_STEPS,),
        in_specs=[
            pl.BlockSpec((_TC_ROWS, 128), lambda t: (t, 0)),
            pl.BlockSpec((_TC_ROWS, 128), lambda t: (0, 0)),
            pl.BlockSpec((128, 128), lambda t: (0, 0)),
        ],
        out_specs=pl.BlockSpec((1, 1), lambda t: (0, 0)),
        out_shape=jax.ShapeDtypeStruct((1, 1), jnp.float32),
    )(scp2, mask_full, g)
    return loss[0, 0]
